# baseline (speedup 1.0000x reference)
import functools

import jax
import jax.numpy as jnp
from jax import lax
from jax.experimental import pallas as pl
from jax.experimental.pallas import tpu as pltpu
from jax.experimental.pallas import tpu_sc as plsc

N = 10000
E = 320000
F_IN = 128
H = 256
KH = 512
G = 64
C = 10

NP = 10240
R = 2048
NBLK = NP // R
NC = 2
NS = 16
CH = 128
EPT = 20480
NCH = EPT // CH
BLK = 32
NRF = NCH // BLK
RPT = NP // NS


def _zero_pad_rows(c):
    i = pl.program_id(0)
    row = lax.broadcasted_iota(jnp.int32, (R, H), 0) + i * R
    return jnp.where(row < N, c, 0.0)


def _tc_first_body(x_ref, w0_ref, b0_ref, w1_ref, b1_ref, w2_ref,
                   ca_ref, cb_ref):
    h = jnp.dot(x_ref[...], w0_ref[...],
                preferred_element_type=jnp.float32) + b0_ref[...]
    m = jnp.maximum(jnp.dot(h, w1_ref[...],
                            preferred_element_type=jnp.float32)
                    + b1_ref[...], 0.0)
    c = jnp.dot(m, w2_ref[...], preferred_element_type=jnp.float32)
    c = _zero_pad_rows(c)
    ca_ref[...] = c[:, :128]
    cb_ref[...] = c[:, 128:]


def _tc_mid_body(ca_ref, cb_ref, aa_ref, ab_ref, b2_ref,
                 w1_ref, b1_ref, w2_ref, oa_ref, ob_ref):
    h = jnp.concatenate([ca_ref[...] + aa_ref[...],
                         cb_ref[...] + ab_ref[...]], axis=1) + b2_ref[...]
    h = jnp.maximum(h, 0.0)
    m = jnp.maximum(jnp.dot(h, w1_ref[...],
                            preferred_element_type=jnp.float32)
                    + b1_ref[...], 0.0)
    c = jnp.dot(m, w2_ref[...], preferred_element_type=jnp.float32)
    c = _zero_pad_rows(c)
    oa_ref[...] = c[:, :128]
    ob_ref[...] = c[:, 128:]


def _tc_final_body(ca_ref, cb_ref, aa_ref, ab_ref, b2_ref, oh_ref,
                   w3_ref, b3_ref, w4_ref, b4_ref, out_ref, p_ref):
    i = pl.program_id(0)
    h = jnp.concatenate([ca_ref[...] + aa_ref[...],
                         cb_ref[...] + ab_ref[...]], axis=1) + b2_ref[...]
    h = jnp.maximum(h, 0.0)
    hh = jnp.concatenate([h, jnp.ones((R, 128), jnp.float32)], axis=1)
    contrib = lax.dot_general(oh_ref[...], hh,
                              (((0,), (0,)), ((), ())),
                              preferred_element_type=jnp.float32)

    @pl.when(i == 0)
    def _():
        p_ref[...] = contrib

    @pl.when(i > 0)
    def _():
        p_ref[...] = p_ref[...] + contrib

    @pl.when(i == NBLK - 1)
    def _():
        seg = p_ref[:G, :H]
        cnt = p_ref[:G, H:H + 1]
        pooled = seg / jnp.maximum(cnt, 1.0)
        z = jnp.maximum(jnp.dot(pooled, w3_ref[...],
                                preferred_element_type=jnp.float32)
                        + b3_ref[...], 0.0)
        logits = jnp.dot(z, w4_ref[...],
                         preferred_element_type=jnp.float32) + b4_ref[...]
        col = lax.broadcasted_iota(jnp.int32, (G, 128), 1)
        masked = jnp.where(col < C, logits, -jnp.inf)
        mx = jnp.max(masked, axis=1, keepdims=True)
        lse = mx + jnp.log(jnp.sum(jnp.where(col < C,
                                             jnp.exp(masked - mx), 0.0),
                                   axis=1, keepdims=True))
        out_ref[...] = logits - lse


_row_spec = pl.BlockSpec((R, 128), lambda i: (i, 0))
_full = lambda shape: pl.BlockSpec(shape, lambda i: tuple(0 for _ in shape))
_half_shape = [jax.ShapeDtypeStruct((NP, 128), jnp.float32)] * 2


def _tc_first(x, w0, b0, w1, b1, w2):
    return pl.pallas_call(
        _tc_first_body,
        grid=(NBLK,),
        in_specs=[
            _row_spec,
            _full((F_IN, H)), _full((1, H)),
            _full((H, KH)), _full((1, KH)),
            _full((KH, H)),
        ],
        out_specs=[_row_spec, _row_spec],
        out_shape=_half_shape,
    )(x, w0, b0, w1, b1, w2)


def _tc_mid(ca, cb, aa, ab, b2, w1, b1, w2):
    return pl.pallas_call(
        _tc_mid_body,
        grid=(NBLK,),
        in_specs=[
            _row_spec, _row_spec, _row_spec, _row_spec,
            _full((1, H)),
            _full((H, KH)), _full((1, KH)),
            _full((KH, H)),
        ],
        out_specs=[_row_spec, _row_spec],
        out_shape=_half_shape,
    )(ca, cb, aa, ab, b2, w1, b1, w2)


def _tc_final(ca, cb, aa, ab, b2, oh, w3, b3, w4, b4):
    return pl.pallas_call(
        _tc_final_body,
        grid=(NBLK,),
        in_specs=[
            _row_spec, _row_spec, _row_spec, _row_spec,
            _full((1, H)),
            _row_spec,
            _full((H, H)), _full((1, H)),
            _full((H, 128)), _full((1, 128)),
        ],
        out_specs=[pl.BlockSpec((G, 128), lambda i: (0, 0))],
        out_shape=[jax.ShapeDtypeStruct((G, 128), jnp.float32)],
        scratch_shapes=[pltpu.VMEM((128, H + 128), jnp.float32)],
    )(ca, cb, aa, ab, b2, oh, w3, b3, w4, b4)[0]


def _sc_agg_body(c_a, c_b, srcp, dstp, zrows, out_a, out_b,
                 srcv, dstv, buf0, buf1, acc, sg0, sg1, ss0, ss1):
    cid = lax.axis_index("c")
    sid = lax.axis_index("s")
    rows = pl.ds(sid * RPT, RPT)
    pltpu.sync_copy(zrows, acc.at[rows])
    plsc.subcore_barrier()

    def run(c_ref):
        def gather(j, buf, sem):
            pltpu.async_copy(c_ref.at[srcv.at[j]], buf, sem)

        def scatter(j, buf, sem):
            pltpu.async_copy(buf, acc.at[dstv.at[j]], sem, add=True)

        def wait(j, buf, sem):
            pltpu.make_async_copy(c_ref.at[srcv.at[j]], buf, sem).wait()

        def wait_sc(j, buf, sem):
            pltpu.make_async_copy(buf, acc.at[dstv.at[j]], sem).wait()

        for b in range(NRF):
            pltpu.sync_copy(srcp.at[sid, pl.ds(b * BLK, BLK)], srcv)
            pltpu.sync_copy(dstp.at[sid, pl.ds(b * BLK, BLK)], dstv)
            gather(0, buf0, sg0)
            gather(1, buf1, sg1)

            def pair(j2, carry):
                j = 2 * j2
                wait(j, buf0, sg0)
                scatter(j, buf0, ss0)
                wait(j + 1, buf1, sg1)
                scatter(j + 1, buf1, ss1)

                @pl.when(j + 2 < BLK)
                def _():
                    wait_sc(j, buf0, ss0)
                    gather(j + 2, buf0, sg0)
                    wait_sc(j + 1, buf1, ss1)
                    gather(j + 3, buf1, sg1)
                return carry
            lax.fori_loop(0, BLK // 2, pair, 0)
            wait_sc(BLK - 2, buf0, ss0)
            wait_sc(BLK - 1, buf1, ss1)

    @pl.when(cid == 0)
    def _():
        run(c_a)

    @pl.when(cid == 1)
    def _():
        run(c_b)

    plsc.subcore_barrier()

    @pl.when(cid == 0)
    def _():
        pltpu.sync_copy(acc.at[rows], out_a.at[rows])

    @pl.when(cid == 1)
    def _():
        pltpu.sync_copy(acc.at[rows], out_b.at[rows])


@functools.cache
def _sc_agg_kernel():
    return pl.kernel(
        _sc_agg_body,
        out_type=_half_shape,
        mesh=plsc.VectorSubcoreMesh(core_axis_name="c",
                                    subcore_axis_name="s"),
        scratch_types=[
            pltpu.VMEM((BLK, CH), jnp.int32),
            pltpu.VMEM((BLK, CH), jnp.int32),
            pltpu.VMEM((CH, 128), jnp.float32),
            pltpu.VMEM((CH, 128), jnp.float32),
            pltpu.VMEM_SHARED((NP, 128), jnp.float32),
            pltpu.SemaphoreType.DMA,
            pltpu.SemaphoreType.DMA,
            pltpu.SemaphoreType.DMA,
            pltpu.SemaphoreType.DMA,
        ],
    )


def _sc_agg(ca, cb, srcp, dstp, zrows):
    return _sc_agg_kernel()(ca, cb, srcp, dstp, zrows)


def kernel(x, edge_index, batch, lin0_W, lin0_b, conv_W1, conv_b1,
           conv_W2, conv_b2, lin1_W, lin1_b, lin2_W, lin2_b):
    src2 = edge_index[0].reshape(NS, E // NS)
    dst2 = edge_index[1].reshape(NS, E // NS)
    pad = EPT - E // NS
    srcp = jnp.pad(src2, ((0, 0), (0, pad)),
                   constant_values=N).reshape(NS, NCH, CH)
    dstp = jnp.pad(dst2, ((0, 0), (0, pad)),
                   constant_values=0).reshape(NS, NCH, CH)
    zrows = jnp.zeros((RPT, 128), jnp.float32)
    x = jnp.pad(x, ((0, NP - N), (0, 0)))
    batch_p = jnp.pad(batch, (0, NP - N), constant_values=127)
    oh = (batch_p[:, None] == jnp.arange(128, dtype=batch.dtype)[None, :]
          ).astype(jnp.float32)
    w4 = jnp.zeros((H, 128), jnp.float32).at[:, :C].set(lin2_W)
    b4 = jnp.zeros((1, 128), jnp.float32).at[0, :C].set(lin2_b)

    ca, cb = _tc_first(x, lin0_W, lin0_b.reshape(1, H),
                       conv_W1[0], conv_b1[0].reshape(1, KH), conv_W2[0])
    for l in range(1, 3):
        aa, ab = _sc_agg(ca, cb, srcp, dstp, zrows)
        ca, cb = _tc_mid(ca, cb, aa, ab, conv_b2[l - 1].reshape(1, H),
                         conv_W1[l], conv_b1[l].reshape(1, KH), conv_W2[l])
    aa, ab = _sc_agg(ca, cb, srcp, dstp, zrows)
    out = _tc_final(ca, cb, aa, ab, conv_b2[2].reshape(1, H), oh,
                    lin1_W, lin1_b.reshape(1, H), w4, b4)
    return out[:, :C]


# speedup vs baseline: 1.1381x; 1.1381x over previous
import functools

import jax
import jax.numpy as jnp
from jax import lax
from jax.experimental import pallas as pl
from jax.experimental.pallas import tpu as pltpu
from jax.experimental.pallas import tpu_sc as plsc

N = 10000
E = 320000
F_IN = 128
H = 256
KH = 512
G = 64
C = 10

NP = 10240
R = 2048
NBLK = NP // R
NC = 2
NS = 16
CH = 64
EPT = 20480
NCH = EPT // CH
BLK = 32
NRF = NCH // BLK
NBUF = 4
RPT = NP // NS


def _zero_pad_rows(c):
    i = pl.program_id(0)
    row = lax.broadcasted_iota(jnp.int32, (R, H), 0) + i * R
    return jnp.where(row < N, c, 0.0)


def _tc_first_body(x_ref, w0_ref, b0_ref, w1_ref, b1_ref, w2_ref,
                   ca_ref, cb_ref):
    h = jnp.dot(x_ref[...], w0_ref[...],
                preferred_element_type=jnp.float32) + b0_ref[...]
    m = jnp.maximum(jnp.dot(h, w1_ref[...],
                            preferred_element_type=jnp.float32)
                    + b1_ref[...], 0.0)
    c = jnp.dot(m, w2_ref[...], preferred_element_type=jnp.float32)
    c = _zero_pad_rows(c)
    ca_ref[...] = c[:, :128]
    cb_ref[...] = c[:, 128:]


def _tc_mid_body(ca_ref, cb_ref, aa_ref, ab_ref, b2_ref,
                 w1_ref, b1_ref, w2_ref, oa_ref, ob_ref):
    h = jnp.concatenate([ca_ref[...] + aa_ref[...],
                         cb_ref[...] + ab_ref[...]], axis=1) + b2_ref[...]
    h = jnp.maximum(h, 0.0)
    m = jnp.maximum(jnp.dot(h, w1_ref[...],
                            preferred_element_type=jnp.float32)
                    + b1_ref[...], 0.0)
    c = jnp.dot(m, w2_ref[...], preferred_element_type=jnp.float32)
    c = _zero_pad_rows(c)
    oa_ref[...] = c[:, :128]
    ob_ref[...] = c[:, 128:]


def _tc_final_body(ca_ref, cb_ref, aa_ref, ab_ref, b2_ref, oh_ref,
                   w3_ref, b3_ref, w4_ref, b4_ref, out_ref, p_ref):
    i = pl.program_id(0)
    h = jnp.concatenate([ca_ref[...] + aa_ref[...],
                         cb_ref[...] + ab_ref[...]], axis=1) + b2_ref[...]
    h = jnp.maximum(h, 0.0)
    hh = jnp.concatenate([h, jnp.ones((R, 128), jnp.float32)], axis=1)
    contrib = lax.dot_general(oh_ref[...], hh,
                              (((0,), (0,)), ((), ())),
                              preferred_element_type=jnp.float32)

    @pl.when(i == 0)
    def _():
        p_ref[...] = contrib

    @pl.when(i > 0)
    def _():
        p_ref[...] = p_ref[...] + contrib

    @pl.when(i == NBLK - 1)
    def _():
        seg = p_ref[:G, :H]
        cnt = p_ref[:G, H:H + 1]
        pooled = seg / jnp.maximum(cnt, 1.0)
        z = jnp.maximum(jnp.dot(pooled, w3_ref[...],
                                preferred_element_type=jnp.float32)
                        + b3_ref[...], 0.0)
        logits = jnp.dot(z, w4_ref[...],
                         preferred_element_type=jnp.float32) + b4_ref[...]
        col = lax.broadcasted_iota(jnp.int32, (G, 128), 1)
        masked = jnp.where(col < C, logits, -jnp.inf)
        mx = jnp.max(masked, axis=1, keepdims=True)
        lse = mx + jnp.log(jnp.sum(jnp.where(col < C,
                                             jnp.exp(masked - mx), 0.0),
                                   axis=1, keepdims=True))
        out_ref[...] = logits - lse


_row_spec = pl.BlockSpec((R, 128), lambda i: (i, 0))
_full = lambda shape: pl.BlockSpec(shape, lambda i: tuple(0 for _ in shape))
_half_shape = [jax.ShapeDtypeStruct((NP, 128), jnp.float32)] * 2


def _tc_first(x, w0, b0, w1, b1, w2):
    return pl.pallas_call(
        _tc_first_body,
        grid=(NBLK,),
        in_specs=[
            _row_spec,
            _full((F_IN, H)), _full((1, H)),
            _full((H, KH)), _full((1, KH)),
            _full((KH, H)),
        ],
        out_specs=[_row_spec, _row_spec],
        out_shape=_half_shape,
    )(x, w0, b0, w1, b1, w2)


def _tc_mid(ca, cb, aa, ab, b2, w1, b1, w2):
    return pl.pallas_call(
        _tc_mid_body,
        grid=(NBLK,),
        in_specs=[
            _row_spec, _row_spec, _row_spec, _row_spec,
            _full((1, H)),
            _full((H, KH)), _full((1, KH)),
            _full((KH, H)),
        ],
        out_specs=[_row_spec, _row_spec],
        out_shape=_half_shape,
    )(ca, cb, aa, ab, b2, w1, b1, w2)


def _tc_final(ca, cb, aa, ab, b2, oh, w3, b3, w4, b4):
    return pl.pallas_call(
        _tc_final_body,
        grid=(NBLK,),
        in_specs=[
            _row_spec, _row_spec, _row_spec, _row_spec,
            _full((1, H)),
            _row_spec,
            _full((H, H)), _full((1, H)),
            _full((H, 128)), _full((1, 128)),
        ],
        out_specs=[pl.BlockSpec((G, 128), lambda i: (0, 0))],
        out_shape=[jax.ShapeDtypeStruct((G, 128), jnp.float32)],
        scratch_shapes=[pltpu.VMEM((128, H + 128), jnp.float32)],
    )(ca, cb, aa, ab, b2, oh, w3, b3, w4, b4)[0]


def _sc_agg_body(c_a, c_b, srcp, dstp, zrows, out_a, out_b,
                 srcv, dstv, buf0, buf1, buf2, buf3, acc,
                 sg0, sg1, sg2, sg3):
    cid = lax.axis_index("c")
    sid = lax.axis_index("s")
    rows = pl.ds(sid * RPT, RPT)
    bufs = (buf0, buf1, buf2, buf3)
    sems = (sg0, sg1, sg2, sg3)
    pltpu.sync_copy(zrows, acc.at[rows])
    plsc.subcore_barrier()

    def run(c_ref):
        def gather(j, k):
            pltpu.async_copy(c_ref.at[srcv.at[j]], bufs[k], sems[k])

        def wait(j, k):
            pltpu.make_async_copy(c_ref.at[srcv.at[j]],
                                  bufs[k], sems[k]).wait()

        for b in range(NRF):
            pltpu.sync_copy(srcp.at[sid, pl.ds(b * BLK, BLK)], srcv)
            pltpu.sync_copy(dstp.at[sid, pl.ds(b * BLK, BLK)], dstv)
            for k in range(NBUF):
                gather(k, k)

            def group(g, carry):
                j = NBUF * g
                for k in range(NBUF):
                    wait(j + k, k)
                    pltpu.sync_copy(bufs[k], acc.at[dstv.at[j + k]],
                                    add=True)

                    @pl.when(j + k + NBUF < BLK)
                    def _():
                        gather(j + k + NBUF, k)
                return carry
            lax.fori_loop(0, BLK // NBUF, group, 0)

    @pl.when(cid == 0)
    def _():
        run(c_a)

    @pl.when(cid == 1)
    def _():
        run(c_b)

    plsc.subcore_barrier()

    @pl.when(cid == 0)
    def _():
        pltpu.sync_copy(acc.at[rows], out_a.at[rows])

    @pl.when(cid == 1)
    def _():
        pltpu.sync_copy(acc.at[rows], out_b.at[rows])


@functools.cache
def _sc_agg_kernel():
    return pl.kernel(
        _sc_agg_body,
        out_type=_half_shape,
        mesh=plsc.VectorSubcoreMesh(core_axis_name="c",
                                    subcore_axis_name="s"),
        scratch_types=[
            pltpu.VMEM((BLK, CH), jnp.int32),
            pltpu.VMEM((BLK, CH), jnp.int32),
            pltpu.VMEM((CH, 128), jnp.float32),
            pltpu.VMEM((CH, 128), jnp.float32),
            pltpu.VMEM((CH, 128), jnp.float32),
            pltpu.VMEM((CH, 128), jnp.float32),
            pltpu.VMEM_SHARED((NP, 128), jnp.float32),
            pltpu.SemaphoreType.DMA,
            pltpu.SemaphoreType.DMA,
            pltpu.SemaphoreType.DMA,
            pltpu.SemaphoreType.DMA,
        ],
    )


def _sc_agg(ca, cb, srcp, dstp, zrows):
    return _sc_agg_kernel()(ca, cb, srcp, dstp, zrows)


def kernel(x, edge_index, batch, lin0_W, lin0_b, conv_W1, conv_b1,
           conv_W2, conv_b2, lin1_W, lin1_b, lin2_W, lin2_b):
    src2 = edge_index[0].reshape(NS, E // NS)
    dst2 = edge_index[1].reshape(NS, E // NS)
    pad = EPT - E // NS
    srcp = jnp.pad(src2, ((0, 0), (0, pad)),
                   constant_values=N).reshape(NS, NCH, CH)
    dstp = jnp.pad(dst2, ((0, 0), (0, pad)),
                   constant_values=0).reshape(NS, NCH, CH)
    zrows = jnp.zeros((RPT, 128), jnp.float32)
    x = jnp.pad(x, ((0, NP - N), (0, 0)))
    batch_p = jnp.pad(batch, (0, NP - N), constant_values=127)
    oh = (batch_p[:, None] == jnp.arange(128, dtype=batch.dtype)[None, :]
          ).astype(jnp.float32)
    w4 = jnp.zeros((H, 128), jnp.float32).at[:, :C].set(lin2_W)
    b4 = jnp.zeros((1, 128), jnp.float32).at[0, :C].set(lin2_b)

    ca, cb = _tc_first(x, lin0_W, lin0_b.reshape(1, H),
                       conv_W1[0], conv_b1[0].reshape(1, KH), conv_W2[0])
    for l in range(1, 3):
        aa, ab = _sc_agg(ca, cb, srcp, dstp, zrows)
        ca, cb = _tc_mid(ca, cb, aa, ab, conv_b2[l - 1].reshape(1, H),
                         conv_W1[l], conv_b1[l].reshape(1, KH), conv_W2[l])
    aa, ab = _sc_agg(ca, cb, srcp, dstp, zrows)
    out = _tc_final(ca, cb, aa, ab, conv_b2[2].reshape(1, H), oh,
                    lin1_W, lin1_b.reshape(1, H), w4, b4)
    return out[:, :C]
